# baseline (speedup 1.0000x reference)
import jax
import jax.numpy as jnp
from jax import lax
from jax.experimental import pallas as pl
from jax.experimental.pallas import tpu as pltpu, tpu_sc as plsc

_B = 16384
_NC, _NS, _L = 2, 16, 16
_NW = _NC * _NS
_BPW = _B // _NW
_P = 64
_NPASS = _BPW // _P


def _body(a_hbm, b_hbm, h_hbm, dw_hbm, cw_hbm, outd_hbm, outc_hbm,
          a_v, b_v, h0_v, h1_v, dw_v, cw_v,
          od0_v, od1_v, oc0_v, oc1_v,
          sem_in, sem_h, sem_out):
    wid = lax.axis_index("s") * _NC + lax.axis_index("c")
    base = wid * _BPW
    h_bufs = (h0_v, h1_v)
    od_bufs = (od0_v, od1_v)
    oc_bufs = (oc0_v, oc1_v)

    cp_a = pltpu.async_copy(a_hbm.at[pl.ds(base, _BPW)], a_v, sem_in)
    cp_b = pltpu.async_copy(b_hbm.at[pl.ds(base, _BPW)], b_v, sem_in)
    cp_dw = pltpu.async_copy(dw_hbm, dw_v, sem_in)
    cp_cw = pltpu.async_copy(cw_hbm, cw_v, sem_in)
    cp_h = [
        pltpu.async_copy(h_hbm.at[pl.ds(base + p * _P, _P)], h_bufs[p], sem_h)
        for p in range(2)
    ]
    cp_a.wait()
    cp_b.wait()
    cp_dw.wait()
    cp_cw.wait()

    lane = lax.iota(jnp.int32, _L)
    par = lane % 2
    m10 = lane < 10
    m2 = lane < 2
    cp_od = [None, None]
    cp_oc = [None, None]
    for p in range(_NPASS):
        buf = p & 1
        h_v = h_bufs[buf]
        outd_v = od_bufs[buf]
        outc_v = oc_bufs[buf]
        cp_h[buf].wait()
        if cp_od[buf] is not None:
            cp_od[buf].wait()
            cp_oc[buf].wait()

        @plsc.parallel_loop(0, _P // _L, unroll=2)
        def chunk(c):
            off = c * _L
            a = a_v[pl.ds(p * _P + off, _L)]
            b = b_v[pl.ds(p * _P + off, _L)]
            row = lane + off
            h_par = plsc.load_gather(h_v, [row, par])
            h_opp = plsc.load_gather(h_v, [row, 1 - par])
            diff = h_opp - h_par
            carry100 = jnp.where(jnp.where(par == 0, diff, -diff) > 0,
                                 100, 0)
            idx16 = carry100 + a * 10 + b
            base_rv = jnp.full((_L,), off, jnp.int32)
            for r2 in range(_L):
                iv = jnp.full((_L,), idx16[r2], jnp.int32)
                rv = base_rv + r2
                dvec = plsc.load_gather(dw_v, [iv, lane], mask=m10)
                plsc.store_scatter(outd_v, [rv, lane], dvec, mask=m10)
                cvec = plsc.load_gather(cw_v, [iv, lane], mask=m2)
                plsc.store_scatter(outc_v, [rv, lane], cvec, mask=m2)

        if p + 2 < _NPASS:
            cp_h[buf] = pltpu.async_copy(
                h_hbm.at[pl.ds(base + (p + 2) * _P, _P)], h_v, sem_h)
        cp_od[buf] = pltpu.async_copy(
            outd_v, outd_hbm.at[pl.ds(base + p * _P, _P)], sem_out)
        cp_oc[buf] = pltpu.async_copy(
            outc_v, outc_hbm.at[pl.ds(base + p * _P, _P)], sem_out)
    for buf in range(2):
        cp_od[buf].wait()
        cp_oc[buf].wait()


@jax.jit
def kernel(a_t, b_t, h_t, next_carry_w, digit_w):
    mesh = plsc.VectorSubcoreMesh(
        core_axis_name="c", subcore_axis_name="s",
        num_cores=_NC, num_subcores=_NS)
    run = pl.kernel(
        _body,
        out_type=(
            jax.ShapeDtypeStruct((_B, 10), jnp.float32),
            jax.ShapeDtypeStruct((_B, 2), jnp.float32),
        ),
        mesh=mesh,
        compiler_params=pltpu.CompilerParams(needs_layout_passes=False),
        scratch_types=[
            pltpu.VMEM((_BPW,), jnp.int32),
            pltpu.VMEM((_BPW,), jnp.int32),
            pltpu.VMEM((_P, 2), jnp.float32),
            pltpu.VMEM((_P, 2), jnp.float32),
            pltpu.VMEM((200, 10), jnp.float32),
            pltpu.VMEM((200, 2), jnp.float32),
            pltpu.VMEM((_P, 10), jnp.float32),
            pltpu.VMEM((_P, 10), jnp.float32),
            pltpu.VMEM((_P, 2), jnp.float32),
            pltpu.VMEM((_P, 2), jnp.float32),
            pltpu.SemaphoreType.DMA,
            pltpu.SemaphoreType.DMA,
            pltpu.SemaphoreType.DMA,
        ],
    )
    return run(a_t.astype(jnp.int32), b_t.astype(jnp.int32),
               h_t, digit_w, next_carry_w)


# speedup vs baseline: 2.2714x; 2.2714x over previous
import jax
import jax.numpy as jnp
from jax import lax
from jax.experimental import pallas as pl
from jax.experimental.pallas import tpu as pltpu, tpu_sc as plsc

_B = 16384
_NC, _NS, _L = 2, 16, 16
_NW = _NC * _NS
_BPW = _B // _NW
_CHUNKS = _BPW // _L


def _body(a_hbm, b_hbm, h_hbm, dw_hbm, cw_hbm, outd_hbm, outc_hbm,
          a_v, b_v, h_v, dw_v, cw_v, outd_v, outc_v, sem_in, sem_out):
    wid = lax.axis_index("s") * _NC + lax.axis_index("c")
    base = wid * _BPW

    cp_a = pltpu.async_copy(a_hbm.at[pl.ds(base, _BPW)], a_v, sem_in)
    cp_b = pltpu.async_copy(b_hbm.at[pl.ds(base, _BPW)], b_v, sem_in)
    cp_h = pltpu.async_copy(h_hbm.at[:, pl.ds(base, _BPW)], h_v, sem_in)
    cp_dw = pltpu.async_copy(dw_hbm, dw_v, sem_in)
    cp_cw = pltpu.async_copy(cw_hbm, cw_v, sem_in)
    cp_a.wait()
    cp_b.wait()
    cp_h.wait()
    cp_dw.wait()
    cp_cw.wait()

    lane = lax.iota(jnp.int32, _L)
    zero = jnp.zeros((_L,), jnp.int32)
    one = zero + 1
    gvecs = [zero + g for g in range(10)]

    @plsc.parallel_loop(0, _CHUNKS, unroll=4)
    def chunk(c):
        off = c * _L
        a = a_v[pl.ds(off, _L)]
        b = b_v[pl.ds(off, _L)]
        row = lane + off
        h0 = plsc.load_gather(h_v, [zero, row])
        h1 = plsc.load_gather(h_v, [one, row])
        carry100 = jnp.where(h1 > h0, 100, 0)
        idx = carry100 + a * 10 + b
        for g in range(10):
            val = plsc.load_gather(dw_v, [gvecs[g], idx])
            plsc.store_scatter(outd_v, [gvecs[g], row], val)
        for g in range(2):
            val = plsc.load_gather(cw_v, [gvecs[g], idx])
            plsc.store_scatter(outc_v, [gvecs[g], row], val)

    cp_od = pltpu.async_copy(outd_v, outd_hbm.at[:, pl.ds(base, _BPW)],
                             sem_out)
    cp_oc = pltpu.async_copy(outc_v, outc_hbm.at[:, pl.ds(base, _BPW)],
                             sem_out)
    cp_od.wait()
    cp_oc.wait()


@jax.jit
def kernel(a_t, b_t, h_t, next_carry_w, digit_w):
    mesh = plsc.VectorSubcoreMesh(
        core_axis_name="c", subcore_axis_name="s",
        num_cores=_NC, num_subcores=_NS)
    run = pl.kernel(
        _body,
        out_type=(
            jax.ShapeDtypeStruct((10, _B), jnp.float32),
            jax.ShapeDtypeStruct((2, _B), jnp.float32),
        ),
        mesh=mesh,
        compiler_params=pltpu.CompilerParams(needs_layout_passes=False),
        scratch_types=[
            pltpu.VMEM((_BPW,), jnp.int32),
            pltpu.VMEM((_BPW,), jnp.int32),
            pltpu.VMEM((2, _BPW), jnp.float32),
            pltpu.VMEM((10, 200), jnp.float32),
            pltpu.VMEM((2, 200), jnp.float32),
            pltpu.VMEM((10, _BPW), jnp.float32),
            pltpu.VMEM((2, _BPW), jnp.float32),
            pltpu.SemaphoreType.DMA,
            pltpu.SemaphoreType.DMA,
        ],
    )
    outd_t, outc_t = run(a_t.astype(jnp.int32), b_t.astype(jnp.int32),
                         h_t.T, digit_w.T, next_carry_w.T)
    return outd_t.T, outc_t.T
